# baseline (speedup 1.0000x reference)
import jax
import jax.numpy as jnp
from jax.experimental import pallas as pl
from jax.experimental.pallas import tpu as pltpu


def _row_block(n: int, cap: int = 400) -> int:
    best = 8
    for b in range(8, cap + 1, 8):
        if n % b == 0:
            best = b
    return best


def _cheb_body(adj_ref, x_ref, w_ref, b_ref, o_ref, tx1_ref):
    p = pl.program_id(0)
    i = pl.program_id(1)
    nb = pl.num_programs(1)
    bm = adj_ref.shape[0]
    base = jnp.where(p == 0, i, nb - 1 - i) * bm

    @pl.when(p == 0)
    def _phase0():
        tx1_ref[pl.ds(base, bm), :] = jnp.dot(
            adj_ref[...], x_ref[...], preferred_element_type=jnp.float32)

    @pl.when(p == 1)
    def _phase1():
        y = jnp.dot(adj_ref[...], tx1_ref[...],
                    preferred_element_type=jnp.float32)
        xb = x_ref[pl.ds(base, bm), :]
        acc = jnp.dot(xb, w_ref[0], preferred_element_type=jnp.float32)
        acc = acc + jnp.dot(tx1_ref[pl.ds(base, bm), :], w_ref[1],
                            preferred_element_type=jnp.float32)
        acc = acc + jnp.dot(2.0 * y - xb, w_ref[2],
                            preferred_element_type=jnp.float32)
        o_ref[pl.ds(base, bm), :] = acc + b_ref[...]


def kernel(x, adj, weight, bias):
    n, d = x.shape
    bm = _row_block(n)
    nb = n // bm
    bias2 = bias.reshape(1, d)

    out = pl.pallas_call(
        _cheb_body,
        grid=(2, nb),
        in_specs=[
            pl.BlockSpec(
                (bm, n),
                lambda p, i: (jnp.where(p == 0, i, nb - 1 - i), 0)),
            pl.BlockSpec((n, d), lambda p, i: (0, 0)),
            pl.BlockSpec(weight.shape, lambda p, i: (0, 0, 0)),
            pl.BlockSpec((1, d), lambda p, i: (0, 0)),
        ],
        out_specs=pl.BlockSpec((n, d), lambda p, i: (0, 0)),
        out_shape=jax.ShapeDtypeStruct((n, d), jnp.float32),
        scratch_shapes=[pltpu.VMEM((n, d), jnp.float32)],
    )(adj, x, weight, bias2)
    return out
